# baseline (speedup 1.0000x reference)
import functools

import jax
import jax.numpy as jnp
from jax import lax
from jax.experimental import pallas as pl
from jax.experimental.pallas import tpu as pltpu
from jax.experimental.pallas import tpu_sc as plsc

N = 10000
NP = 10240
H = 128
E = 320000
NC = 2
NS = 16
NW = NC * NS
EPW = E // NS
CH = 80
NCH = EPW // CH
RPT = NP // NS
CW = 128
CWT = 8


@functools.cache
def _make_seg_sum():
    mesh = plsc.VectorSubcoreMesh(core_axis_name="c", subcore_axis_name="s",
                                  num_cores=NC, num_subcores=NS)
    return pl.kernel(
        _seg_sum_body,
        out_type=jax.ShapeDtypeStruct((NC, NP, H), jnp.float32),
        mesh=mesh,
        scratch_types=(
            [pltpu.VMEM((CH,), jnp.int32) for _ in range(16)]
            + [pltpu.VMEM((CH, H), jnp.float32) for _ in range(4)]
            + [
                pltpu.VMEM_SHARED((NP, H), jnp.float32),
                pltpu.SemaphoreType.DMA,
                pltpu.SemaphoreType.DMA,
                pltpu.SemaphoreType.DMA,
            ]
        ),
    )


def _seg_sum(table, src, dst):
    return _make_seg_sum()(table, src, dst)


def _seg_sum_body(tab_hbm, src_hbm, dst_hbm, out_hbm,
                  si0, si1, si2, si3, si4, si5, si6, si7,
                  dc0, dc1, dc2, dc3, dc4, dc5, dc6, dc7,
                  rows0, rows1, rows2, rows3, acc, gsem, isem, ssem):
    sis = (si0, si1, si2, si3, si4, si5, si6, si7)
    dcs = (dc0, dc1, dc2, dc3, dc4, dc5, dc6, dc7)
    rows = (rows0, rows1, rows2, rows3)
    c = lax.axis_index("c")
    s = lax.axis_index("s")
    wid = s * NC + c
    ebase = wid * EPW

    zero = jnp.zeros((16,), jnp.float32)

    def zrow(i, carry):
        for j in range(H // 16):
            rows0[i, pl.ds(j * 16, 16)] = zero
        return carry

    lax.fori_loop(0, CH, zrow, 0)
    for t in range(RPT // CH):
        pltpu.sync_copy(rows0, acc.at[pl.ds(s * RPT + t * CH, CH)])
    plsc.subcore_barrier()

    def fire_idx(ch, b):
        pltpu.async_copy(src_hbm.at[pl.ds(ebase + ch * CH, CH)], sis[b], isem)
        pltpu.async_copy(dst_hbm.at[pl.ds(ebase + ch * CH, CH)], dcs[b], isem)

    def wait_idx(b):
        pltpu.make_async_copy(src_hbm.at[pl.ds(0, CH)], sis[b], isem).wait()
        pltpu.make_async_copy(dst_hbm.at[pl.ds(0, CH)], dcs[b], isem).wait()

    def fire_gather(r, b):
        pltpu.async_copy(tab_hbm.at[sis[b]], rows[r], gsem)

    def wait_gather(r):
        pltpu.make_async_copy(tab_hbm.at[sis[0]], rows[r], gsem).wait()

    def fire_scat(r, b):
        pltpu.async_copy(rows[r], acc.at[dcs[b]], ssem, add=True)

    def wait_scat():
        pltpu.make_async_copy(rows[0], acc.at[dcs[0]], ssem).wait()

    def pair(p, q):
        ch = 2 * p
        if isinstance(p, int):
            g2 = 2 * p + 3 < NCH
            i2 = 2 * p + 5 < NCH
        else:
            g2 = i2 = True
        a, b_ = (2 * q) % 4, (2 * q + 1) % 4
        na, nb = (2 * q + 2) % 4, (2 * q + 3) % 4
        da, db = (2 * q) % 8, (2 * q + 1) % 8
        ia, ib = (2 * q + 2) % 8, (2 * q + 3) % 8
        fa, fb = (2 * q + 4) % 8, (2 * q + 5) % 8
        wait_gather(a)
        wait_gather(b_)
        if g2:
            wait_idx(ia)
            wait_idx(ib)
            fire_gather(na, ia)
            fire_gather(nb, ib)
        fire_scat(a, da)
        fire_scat(b_, db)
        if i2:
            fire_idx(ch + 4, fa)
            fire_idx(ch + 5, fb)
        wait_scat()
        wait_scat()

    fire_idx(0, 0)
    fire_idx(1, 1)
    fire_idx(2, 2)
    fire_idx(3, 3)
    wait_idx(0)
    wait_idx(1)
    fire_gather(0, 0)
    fire_gather(1, 1)

    def body(i, carry):
        p0 = 4 * i
        for q in range(4):
            pair(p0 + q, q)
        return carry

    NPAIR = NCH // 2
    KP = (NPAIR - 3) // 4
    lax.fori_loop(0, KP, body, 0)
    for p in range(4 * KP, NPAIR):
        pair(p, p % 4)

    plsc.subcore_barrier()
    pltpu.sync_copy(acc.at[pl.ds(s * RPT, RPT)], out_hbm.at[c, pl.ds(s * RPT, RPT)])


@functools.cache
def _make_seg_count():
    mesh = plsc.VectorSubcoreMesh(core_axis_name="c", subcore_axis_name="s",
                                  num_cores=NC, num_subcores=NS)
    return pl.kernel(
        _seg_count_body,
        out_type=jax.ShapeDtypeStruct((NC, NP, CW), jnp.float32),
        mesh=mesh,
        scratch_types=[
            pltpu.VMEM((EPW,), jnp.int32),
            pltpu.VMEM((CH,), jnp.int32),
            pltpu.VMEM((CH,), jnp.int32),
            pltpu.VMEM((CH, CW), jnp.float32),
            pltpu.VMEM_SHARED((NP, CW), jnp.float32),
            pltpu.SemaphoreType.DMA,
        ],
    )


def _seg_count(dst):
    ones = jnp.ones((CH, CW), jnp.float32)
    zeros = jnp.zeros((RPT, CW), jnp.float32)
    return _make_seg_count()(dst, ones, zeros)


def _seg_count_body(dst_hbm, ones_hbm, zeros_hbm, out_hbm, didx, dc0, dc1,
                    ones, acc, sem):
    c = lax.axis_index("c")
    s = lax.axis_index("s")
    wid = s * NC + c

    pltpu.sync_copy(dst_hbm.at[pl.ds(wid * EPW, EPW)], didx)
    pltpu.sync_copy(ones_hbm, ones)
    pltpu.sync_copy(zeros_hbm, acc.at[pl.ds(s * RPT, RPT)])
    plsc.subcore_barrier()

    def stage(ch, dc):
        for j in range(CH // 16):
            dc[pl.ds(j * 16, 16)] = didx[pl.ds(ch * CH + j * 16, 16)]

    def fire(dc):
        pltpu.async_copy(ones, acc.at[dc], sem, add=True)

    def wait_one(dc):
        pltpu.make_async_copy(ones, acc.at[dc], sem).wait()

    stage(0, dc0)
    fire(dc0)

    def body(i, carry):
        ch = 2 * i
        stage(ch + 1, dc1)
        fire(dc1)
        wait_one(dc0)
        stage(ch + 2, dc0)
        fire(dc0)
        wait_one(dc1)
        return carry

    lax.fori_loop(0, (NCH - 2) // 2, body, 0)
    stage(NCH - 1, dc1)
    fire(dc1)
    wait_one(dc0)
    wait_one(dc1)
    plsc.subcore_barrier()
    pltpu.sync_copy(acc.at[pl.ds(s * RPT, RPT)], out_hbm.at[c, pl.ds(s * RPT, RPT)])


_BR = 1280
_NB = NP // _BR


def _row_spec():
    return pl.BlockSpec((_BR, H), lambda i: (i, 0))


def _cnt_spec():
    return pl.BlockSpec((_BR, CWT), lambda i: (i, 0))


def _wstack_spec():
    return pl.BlockSpec((1, H, H), lambda i: (i // _NB, 0, 0))


def _bstack_spec():
    return pl.BlockSpec((1, 1, H), lambda i: (i // _NB, 0, 0))


def _w_spec():
    return pl.BlockSpec((H, H), lambda i: (0, 0))


def _b_spec():
    return pl.BlockSpec((1, H), lambda i: (0, 0))


def _dot(a, b):
    return jnp.dot(a, b, preferred_element_type=jnp.float32,
                   precision=lax.Precision.HIGHEST)


def _in_proj_body(x, a, cv, o):
    o[...] = jnp.maximum(_dot(x[...], a[...][0]) + cv[...][0], 0.0)


def _in_proj(x2, a2, c2):
    return pl.pallas_call(
        _in_proj_body,
        grid=(2 * _NB,),
        in_specs=[_row_spec(), _wstack_spec(), _bstack_spec()],
        out_specs=_row_spec(),
        out_shape=jax.ShapeDtypeStruct((2 * NP, H), jnp.float32),
    )(x2, a2, c2)


def _sage_update(agg, cnt, h, al, ar, cv):
    inv = 1.0 / jnp.maximum(cnt[...][:, :1], 1.0)
    mean = agg[...] * inv
    hh = h[...]
    z = _dot(mean, al[...][0]) + _dot(hh, ar[...][0]) + cv[...][0]
    return jnp.maximum(z, 0.0) + hh


def _layer_mid_body(agg, cnt, h, al, ar, cv, o):
    o[...] = _sage_update(agg, cnt, h, al, ar, cv)


def _layer_fin_body(agg, cnt, h, al, ar, cv, wf, bf, o):
    o[...] = _dot(_sage_update(agg, cnt, h, al, ar, cv), wf[...]) + bf[...]


def _layer(agg2, cnt2, h2, al2, ar2, cv2, fin=None):
    in_specs = [_row_spec(), _cnt_spec(), _row_spec(),
                _wstack_spec(), _wstack_spec(), _bstack_spec()]
    args = [agg2, cnt2, h2, al2, ar2, cv2]
    if fin is None:
        body = _layer_mid_body
    else:
        body = _layer_fin_body
        in_specs = in_specs + [_w_spec(), _b_spec()]
        args = args + list(fin)
    return pl.pallas_call(
        body,
        grid=(2 * _NB,),
        in_specs=in_specs,
        out_specs=_row_spec(),
        out_shape=jax.ShapeDtypeStruct((2 * NP, H), jnp.float32),
    )(*args)


_BN_S = 1.0 / jnp.sqrt(jnp.float32(1.0 + 1e-5))


def kernel(x_user, x_item, edge_index_user_to_item, edge_index_item_to_user,
           params):
    src_iu = edge_index_item_to_user[0] + NP
    dst_iu = edge_index_item_to_user[1]
    src_ui = edge_index_user_to_item[0]
    dst_ui = edge_index_user_to_item[1]

    def comb(a_iu, a_ui):
        return jnp.stack([a_iu.reshape(NS, EPW), a_ui.reshape(NS, EPW)],
                         axis=1).reshape(NW * EPW)

    src_c = comb(src_iu, src_ui)
    dst_c = comb(dst_iu, dst_ui)

    pad = ((0, NP - N), (0, 0))
    x2 = jnp.concatenate([jnp.pad(x_user, pad), jnp.pad(x_item, pad)])

    cnts = _seg_count(dst_c)
    cnt2 = cnts[:, :, :CWT].reshape(2 * NP, CWT)

    def folded(W, b, w2, b2):
        s = w2 * _BN_S
        return W.T * s[None, :], (b * s + b2)[None, :]

    au, cu = folded(*params["lin_in"]["user"], *params["bn_in"]["user"])
    ai, ci = folded(*params["lin_in"]["item"], *params["bn_in"]["item"])
    h2 = _in_proj(x2, jnp.stack([au, ai]), jnp.stack([cu, ci]))

    Wf, bf = params["final"]
    n_layers = len(params["layers"])
    for li, layer in enumerate(params["layers"]):
        agg = _seg_sum(h2, src_c, dst_c)
        als, ars, cvs = [], [], []
        for nt, conv_key in (("user", "item_to_user"),
                             ("item", "user_to_item")):
            Wl, bl, Wr = layer["conv"][conv_key]
            w2, b2 = layer["bn"][nt]
            s = w2 * _BN_S
            als.append(Wl.T * s[None, :])
            ars.append(Wr.T * s[None, :])
            cvs.append((bl * s + b2)[None, :])
        fin = (Wf.T, bf[None, :]) if li == n_layers - 1 else None
        h2 = _layer(agg.reshape(2 * NP, H), cnt2, h2,
                    jnp.stack(als), jnp.stack(ars), jnp.stack(cvs), fin)

    return (h2[:N], h2[NP:NP + N])


# speedup vs baseline: 1.1810x; 1.1810x over previous
import functools

import jax
import jax.numpy as jnp
from jax import lax
from jax.experimental import pallas as pl
from jax.experimental.pallas import tpu as pltpu
from jax.experimental.pallas import tpu_sc as plsc

N = 10000
NP = 10240
H = 128
E = 320000
NC = 2
NS = 16
NW = NC * NS
EPW = E // NS
CH = 80
NCH = EPW // CH
RPT = NP // NS
CW = 128
CWT = 8


@functools.cache
def _make_seg_sum():
    mesh = plsc.VectorSubcoreMesh(core_axis_name="c", subcore_axis_name="s",
                                  num_cores=NC, num_subcores=NS)
    return pl.kernel(
        _seg_sum_body,
        out_type=jax.ShapeDtypeStruct((NC, NP, H), jnp.float32),
        mesh=mesh,
        scratch_types=(
            [pltpu.VMEM((CH,), jnp.int32) for _ in range(16)]
            + [pltpu.VMEM((CH, H), jnp.float32) for _ in range(4)]
            + [
                pltpu.VMEM_SHARED((NP, H), jnp.float32),
                pltpu.SemaphoreType.DMA,
                pltpu.SemaphoreType.DMA,
                pltpu.SemaphoreType.DMA,
            ]
        ),
    )


def _seg_sum(table, src, dst):
    return _make_seg_sum()(table, src, dst)


def _seg_sum_body(tab_hbm, src_hbm, dst_hbm, out_hbm,
                  si0, si1, si2, si3, si4, si5, si6, si7,
                  dc0, dc1, dc2, dc3, dc4, dc5, dc6, dc7,
                  rows0, rows1, rows2, rows3, acc, gsem, isem, ssem):
    sis = (si0, si1, si2, si3, si4, si5, si6, si7)
    dcs = (dc0, dc1, dc2, dc3, dc4, dc5, dc6, dc7)
    rows = (rows0, rows1, rows2, rows3)
    c = lax.axis_index("c")
    s = lax.axis_index("s")
    wid = s * NC + c
    ebase = wid * EPW

    zero = jnp.zeros((16,), jnp.float32)

    def zrow(i, carry):
        for j in range(H // 16):
            rows0[i, pl.ds(j * 16, 16)] = zero
        return carry

    lax.fori_loop(0, CH, zrow, 0)
    for t in range(RPT // CH):
        pltpu.sync_copy(rows0, acc.at[pl.ds(s * RPT + t * CH, CH)])
    plsc.subcore_barrier()

    def fire_idx(ch, b):
        pltpu.async_copy(src_hbm.at[pl.ds(ebase + ch * CH, CH)], sis[b], isem)
        pltpu.async_copy(dst_hbm.at[pl.ds(ebase + ch * CH, CH)], dcs[b], isem)

    def wait_idx(b):
        pltpu.make_async_copy(src_hbm.at[pl.ds(0, CH)], sis[b], isem).wait()
        pltpu.make_async_copy(dst_hbm.at[pl.ds(0, CH)], dcs[b], isem).wait()

    def fire_gather(r, b):
        pltpu.async_copy(tab_hbm.at[sis[b]], rows[r], gsem)

    def wait_gather(r):
        pltpu.make_async_copy(tab_hbm.at[sis[0]], rows[r], gsem).wait()

    def scat(r, b):
        pltpu.sync_copy(rows[r], acc.at[dcs[b]], add=True)

    def slot(ch, j):
        if isinstance(ch, int):
            has_gather = ch + 3 < NCH
            has_idx = ch + 4 < NCH
        else:
            has_gather = has_idx = True
        if has_gather:
            wait_idx((j + 3) % 8)
            fire_gather((j + 3) % 4, (j + 3) % 8)
        wait_gather(j % 4)
        if has_idx:
            fire_idx(ch + 4, (j + 4) % 8)
        scat(j % 4, j % 8)

    fire_idx(0, 0)
    fire_idx(1, 1)
    fire_idx(2, 2)
    fire_idx(3, 3)
    wait_idx(0)
    fire_gather(0, 0)
    wait_idx(1)
    fire_gather(1, 1)
    wait_idx(2)
    fire_gather(2, 2)

    def body(i, carry):
        ch = 8 * i
        for j in range(8):
            slot(ch + j, j)
        return carry

    K = (NCH - 4) // 8
    lax.fori_loop(0, K, body, 0)
    for ch in range(8 * K, NCH):
        slot(ch, ch % 8)

    plsc.subcore_barrier()
    pltpu.sync_copy(acc.at[pl.ds(s * RPT, RPT)], out_hbm.at[c, pl.ds(s * RPT, RPT)])


@functools.cache
def _make_seg_count():
    mesh = plsc.VectorSubcoreMesh(core_axis_name="c", subcore_axis_name="s",
                                  num_cores=NC, num_subcores=NS)
    return pl.kernel(
        _seg_count_body,
        out_type=jax.ShapeDtypeStruct((NC, NP, CW), jnp.float32),
        mesh=mesh,
        scratch_types=[
            pltpu.VMEM((EPW,), jnp.int32),
            pltpu.VMEM((CH,), jnp.int32),
            pltpu.VMEM((CH,), jnp.int32),
            pltpu.VMEM((CH, CW), jnp.float32),
            pltpu.VMEM_SHARED((NP, CW), jnp.float32),
            pltpu.SemaphoreType.DMA,
        ],
    )


def _seg_count(dst):
    ones = jnp.ones((CH, CW), jnp.float32)
    zeros = jnp.zeros((RPT, CW), jnp.float32)
    return _make_seg_count()(dst, ones, zeros)


def _seg_count_body(dst_hbm, ones_hbm, zeros_hbm, out_hbm, didx, dc0, dc1,
                    ones, acc, sem):
    c = lax.axis_index("c")
    s = lax.axis_index("s")
    wid = s * NC + c

    pltpu.sync_copy(dst_hbm.at[pl.ds(wid * EPW, EPW)], didx)
    pltpu.sync_copy(ones_hbm, ones)
    pltpu.sync_copy(zeros_hbm, acc.at[pl.ds(s * RPT, RPT)])
    plsc.subcore_barrier()

    def stage(ch, dc):
        for j in range(CH // 16):
            dc[pl.ds(j * 16, 16)] = didx[pl.ds(ch * CH + j * 16, 16)]

    def fire(dc):
        pltpu.async_copy(ones, acc.at[dc], sem, add=True)

    def wait_one(dc):
        pltpu.make_async_copy(ones, acc.at[dc], sem).wait()

    stage(0, dc0)
    fire(dc0)

    def body(i, carry):
        ch = 2 * i
        stage(ch + 1, dc1)
        fire(dc1)
        wait_one(dc0)
        stage(ch + 2, dc0)
        fire(dc0)
        wait_one(dc1)
        return carry

    lax.fori_loop(0, (NCH - 2) // 2, body, 0)
    stage(NCH - 1, dc1)
    fire(dc1)
    wait_one(dc0)
    wait_one(dc1)
    plsc.subcore_barrier()
    pltpu.sync_copy(acc.at[pl.ds(s * RPT, RPT)], out_hbm.at[c, pl.ds(s * RPT, RPT)])


_BR = 1280
_NB = NP // _BR


def _row_spec():
    return pl.BlockSpec((_BR, H), lambda i: (i, 0))


def _cnt_spec():
    return pl.BlockSpec((_BR, CWT), lambda i: (i, 0))


def _wstack_spec():
    return pl.BlockSpec((1, H, H), lambda i: (i // _NB, 0, 0))


def _bstack_spec():
    return pl.BlockSpec((1, 1, H), lambda i: (i // _NB, 0, 0))


def _w_spec():
    return pl.BlockSpec((H, H), lambda i: (0, 0))


def _b_spec():
    return pl.BlockSpec((1, H), lambda i: (0, 0))


def _dot(a, b):
    return jnp.dot(a, b, preferred_element_type=jnp.float32,
                   precision=lax.Precision.HIGHEST)


def _in_proj_body(x, a, cv, o):
    o[...] = jnp.maximum(_dot(x[...], a[...][0]) + cv[...][0], 0.0)


def _in_proj(x2, a2, c2):
    return pl.pallas_call(
        _in_proj_body,
        grid=(2 * _NB,),
        in_specs=[_row_spec(), _wstack_spec(), _bstack_spec()],
        out_specs=_row_spec(),
        out_shape=jax.ShapeDtypeStruct((2 * NP, H), jnp.float32),
    )(x2, a2, c2)


def _sage_update(agg, cnt, h, al, ar, cv):
    inv = 1.0 / jnp.maximum(cnt[...][:, :1], 1.0)
    mean = agg[...] * inv
    hh = h[...]
    z = _dot(mean, al[...][0]) + _dot(hh, ar[...][0]) + cv[...][0]
    return jnp.maximum(z, 0.0) + hh


def _layer_mid_body(agg, cnt, h, al, ar, cv, o):
    o[...] = _sage_update(agg, cnt, h, al, ar, cv)


def _layer_fin_body(agg, cnt, h, al, ar, cv, wf, bf, o):
    o[...] = _dot(_sage_update(agg, cnt, h, al, ar, cv), wf[...]) + bf[...]


def _layer(agg2, cnt2, h2, al2, ar2, cv2, fin=None):
    in_specs = [_row_spec(), _cnt_spec(), _row_spec(),
                _wstack_spec(), _wstack_spec(), _bstack_spec()]
    args = [agg2, cnt2, h2, al2, ar2, cv2]
    if fin is None:
        body = _layer_mid_body
    else:
        body = _layer_fin_body
        in_specs = in_specs + [_w_spec(), _b_spec()]
        args = args + list(fin)
    return pl.pallas_call(
        body,
        grid=(2 * _NB,),
        in_specs=in_specs,
        out_specs=_row_spec(),
        out_shape=jax.ShapeDtypeStruct((2 * NP, H), jnp.float32),
    )(*args)


_BN_S = 1.0 / jnp.sqrt(jnp.float32(1.0 + 1e-5))


def kernel(x_user, x_item, edge_index_user_to_item, edge_index_item_to_user,
           params):
    src_iu = edge_index_item_to_user[0] + NP
    dst_iu = edge_index_item_to_user[1]
    src_ui = edge_index_user_to_item[0]
    dst_ui = edge_index_user_to_item[1]

    def comb(a_iu, a_ui):
        return jnp.stack([a_iu.reshape(NS, EPW), a_ui.reshape(NS, EPW)],
                         axis=1).reshape(NW * EPW)

    src_c = comb(src_iu, src_ui)
    dst_c = comb(dst_iu, dst_ui)

    pad = ((0, NP - N), (0, 0))
    x2 = jnp.concatenate([jnp.pad(x_user, pad), jnp.pad(x_item, pad)])

    cnts = _seg_count(dst_c)
    cnt2 = cnts[:, :, :CWT].reshape(2 * NP, CWT)

    def folded(W, b, w2, b2):
        s = w2 * _BN_S
        return W.T * s[None, :], (b * s + b2)[None, :]

    au, cu = folded(*params["lin_in"]["user"], *params["bn_in"]["user"])
    ai, ci = folded(*params["lin_in"]["item"], *params["bn_in"]["item"])
    h2 = _in_proj(x2, jnp.stack([au, ai]), jnp.stack([cu, ci]))

    Wf, bf = params["final"]
    n_layers = len(params["layers"])
    for li, layer in enumerate(params["layers"]):
        agg = _seg_sum(h2, src_c, dst_c)
        als, ars, cvs = [], [], []
        for nt, conv_key in (("user", "item_to_user"),
                             ("item", "user_to_item")):
            Wl, bl, Wr = layer["conv"][conv_key]
            w2, b2 = layer["bn"][nt]
            s = w2 * _BN_S
            als.append(Wl.T * s[None, :])
            ars.append(Wr.T * s[None, :])
            cvs.append((bl * s + b2)[None, :])
        fin = (Wf.T, bf[None, :]) if li == n_layers - 1 else None
        h2 = _layer(agg.reshape(2 * NP, H), cnt2, h2,
                    jnp.stack(als), jnp.stack(ars), jnp.stack(cvs), fin)

    return (h2[:N], h2[NP:NP + N])


# speedup vs baseline: 1.1816x; 1.0005x over previous
import functools

import jax
import jax.numpy as jnp
from jax import lax
from jax.experimental import pallas as pl
from jax.experimental.pallas import tpu as pltpu
from jax.experimental.pallas import tpu_sc as plsc

N = 10000
NP = 10240
H = 128
E = 320000
NC = 2
NS = 16
NW = NC * NS
EPW = E // NS
CH = 80
NCH = EPW // CH
RPT = NP // NS
CW = 128
CWT = 8


@functools.cache
def _make_seg_sum():
    mesh = plsc.VectorSubcoreMesh(core_axis_name="c", subcore_axis_name="s",
                                  num_cores=NC, num_subcores=NS)
    return pl.kernel(
        _seg_sum_body,
        out_type=jax.ShapeDtypeStruct((NC, NP, H), jnp.float32),
        mesh=mesh,
        scratch_types=(
            [pltpu.VMEM((CH,), jnp.int32) for _ in range(16)]
            + [pltpu.VMEM((CH, H), jnp.float32) for _ in range(4)]
            + [
                pltpu.VMEM_SHARED((NP, H), jnp.float32),
                pltpu.SemaphoreType.DMA,
                pltpu.SemaphoreType.DMA,
            ]
        ),
    )


def _seg_sum(table, src, dst):
    return _make_seg_sum()(table, src, dst)


def _seg_sum_body(tab_hbm, src_hbm, dst_hbm, out_hbm,
                  si0, si1, si2, si3, si4, si5, si6, si7,
                  dc0, dc1, dc2, dc3, dc4, dc5, dc6, dc7,
                  rows0, rows1, rows2, rows3, acc, gsem, isem):
    sis = (si0, si1, si2, si3, si4, si5, si6, si7)
    dcs = (dc0, dc1, dc2, dc3, dc4, dc5, dc6, dc7)
    rows = (rows0, rows1, rows2, rows3)
    c = lax.axis_index("c")
    s = lax.axis_index("s")
    wid = s * NC + c
    ebase = wid * EPW

    zero = jnp.zeros((16,), jnp.float32)

    def zrow(i, carry):
        for j in range(H // 16):
            rows0[i, pl.ds(j * 16, 16)] = zero
        return carry

    lax.fori_loop(0, CH, zrow, 0)
    for t in range(RPT // CH):
        pltpu.sync_copy(rows0, acc.at[pl.ds(s * RPT + t * CH, CH)])
    plsc.subcore_barrier()

    def fire_idx(ch, b):
        pltpu.async_copy(src_hbm.at[pl.ds(ebase + ch * CH, CH)], sis[b], isem)
        pltpu.async_copy(dst_hbm.at[pl.ds(ebase + ch * CH, CH)], dcs[b], isem)

    def wait_idx(b):
        pltpu.make_async_copy(src_hbm.at[pl.ds(0, CH)], sis[b], isem).wait()
        pltpu.make_async_copy(dst_hbm.at[pl.ds(0, CH)], dcs[b], isem).wait()

    def fire_gather(r, b):
        pltpu.async_copy(tab_hbm.at[sis[b]], rows[r], gsem)

    def wait_gather(r):
        pltpu.make_async_copy(tab_hbm.at[sis[0]], rows[r], gsem).wait()

    def scat(r, b):
        pltpu.sync_copy(rows[r], acc.at[dcs[b]], add=True)

    def slot(ch, j):
        if isinstance(ch, int):
            has_gather = ch + 3 < NCH
            has_idx = ch + 4 < NCH
        else:
            has_gather = has_idx = True
        if has_gather:
            wait_idx((j + 3) % 8)
            fire_gather((j + 3) % 4, (j + 3) % 8)
        wait_gather(j % 4)
        if has_idx:
            fire_idx(ch + 4, (j + 4) % 8)
        scat(j % 4, j % 8)

    fire_idx(0, 0)
    fire_idx(1, 1)
    fire_idx(2, 2)
    fire_idx(3, 3)
    wait_idx(0)
    fire_gather(0, 0)
    wait_idx(1)
    fire_gather(1, 1)
    wait_idx(2)
    fire_gather(2, 2)

    def body(i, carry):
        ch = 8 * i
        for j in range(8):
            slot(ch + j, j)
        return carry

    K = (NCH - 4) // 8
    lax.fori_loop(0, K, body, 0)
    for ch in range(8 * K, NCH):
        slot(ch, ch % 8)

    plsc.subcore_barrier()
    pltpu.sync_copy(acc.at[pl.ds(s * RPT, RPT)], out_hbm.at[c, pl.ds(s * RPT, RPT)])


@functools.cache
def _make_seg_count():
    mesh = plsc.VectorSubcoreMesh(core_axis_name="c", subcore_axis_name="s",
                                  num_cores=NC, num_subcores=NS)
    return pl.kernel(
        _seg_count_body,
        out_type=jax.ShapeDtypeStruct((NC, NP, CW), jnp.float32),
        mesh=mesh,
        scratch_types=[
            pltpu.VMEM((EPW,), jnp.int32),
            pltpu.VMEM((CH,), jnp.int32),
            pltpu.VMEM((CH,), jnp.int32),
            pltpu.VMEM((CH, CW), jnp.float32),
            pltpu.VMEM_SHARED((NP, CW), jnp.float32),
            pltpu.SemaphoreType.DMA,
        ],
    )


def _seg_count(dst):
    ones = jnp.ones((CH, CW), jnp.float32)
    zeros = jnp.zeros((RPT, CW), jnp.float32)
    return _make_seg_count()(dst, ones, zeros)


def _seg_count_body(dst_hbm, ones_hbm, zeros_hbm, out_hbm, didx, dc0, dc1,
                    ones, acc, sem):
    c = lax.axis_index("c")
    s = lax.axis_index("s")
    wid = s * NC + c

    pltpu.sync_copy(dst_hbm.at[pl.ds(wid * EPW, EPW)], didx)
    pltpu.sync_copy(ones_hbm, ones)
    pltpu.sync_copy(zeros_hbm, acc.at[pl.ds(s * RPT, RPT)])
    plsc.subcore_barrier()

    def stage(ch, dc):
        for j in range(CH // 16):
            dc[pl.ds(j * 16, 16)] = didx[pl.ds(ch * CH + j * 16, 16)]

    def fire(dc):
        pltpu.async_copy(ones, acc.at[dc], sem, add=True)

    def wait_one(dc):
        pltpu.make_async_copy(ones, acc.at[dc], sem).wait()

    stage(0, dc0)
    fire(dc0)

    def body(i, carry):
        ch = 2 * i
        stage(ch + 1, dc1)
        fire(dc1)
        wait_one(dc0)
        stage(ch + 2, dc0)
        fire(dc0)
        wait_one(dc1)
        return carry

    lax.fori_loop(0, (NCH - 2) // 2, body, 0)
    stage(NCH - 1, dc1)
    fire(dc1)
    wait_one(dc0)
    wait_one(dc1)
    plsc.subcore_barrier()
    pltpu.sync_copy(acc.at[pl.ds(s * RPT, RPT)], out_hbm.at[c, pl.ds(s * RPT, RPT)])


_BR = 1280
_NB = NP // _BR


def _row_spec():
    return pl.BlockSpec((_BR, H), lambda i: (i, 0))


def _cnt_spec():
    return pl.BlockSpec((_BR, CWT), lambda i: (i, 0))


def _wstack_spec():
    return pl.BlockSpec((1, H, H), lambda i: (i // _NB, 0, 0))


def _bstack_spec():
    return pl.BlockSpec((1, 1, H), lambda i: (i // _NB, 0, 0))


def _w_spec():
    return pl.BlockSpec((H, H), lambda i: (0, 0))


def _b_spec():
    return pl.BlockSpec((1, H), lambda i: (0, 0))


def _dot(a, b):
    return jnp.dot(a, b, preferred_element_type=jnp.float32,
                   precision=lax.Precision.HIGHEST)


def _in_proj_body(x, a, cv, o):
    o[...] = jnp.maximum(_dot(x[...], a[...][0]) + cv[...][0], 0.0)


def _in_proj(x2, a2, c2):
    return pl.pallas_call(
        _in_proj_body,
        grid=(2 * _NB,),
        in_specs=[_row_spec(), _wstack_spec(), _bstack_spec()],
        out_specs=_row_spec(),
        out_shape=jax.ShapeDtypeStruct((2 * NP, H), jnp.float32),
    )(x2, a2, c2)


def _sage_update(agg, cnt, h, al, ar, cv):
    inv = 1.0 / jnp.maximum(cnt[...][:, :1], 1.0)
    mean = agg[...] * inv
    hh = h[...]
    z = _dot(mean, al[...][0]) + _dot(hh, ar[...][0]) + cv[...][0]
    return jnp.maximum(z, 0.0) + hh


def _layer_mid_body(agg, cnt, h, al, ar, cv, o):
    o[...] = _sage_update(agg, cnt, h, al, ar, cv)


def _layer_fin_body(agg, cnt, h, al, ar, cv, wf, bf, o):
    o[...] = _dot(_sage_update(agg, cnt, h, al, ar, cv), wf[...]) + bf[...]


def _layer(agg2, cnt2, h2, al2, ar2, cv2, fin=None):
    in_specs = [_row_spec(), _cnt_spec(), _row_spec(),
                _wstack_spec(), _wstack_spec(), _bstack_spec()]
    args = [agg2, cnt2, h2, al2, ar2, cv2]
    if fin is None:
        body = _layer_mid_body
    else:
        body = _layer_fin_body
        in_specs = in_specs + [_w_spec(), _b_spec()]
        args = args + list(fin)
    return pl.pallas_call(
        body,
        grid=(2 * _NB,),
        in_specs=in_specs,
        out_specs=_row_spec(),
        out_shape=jax.ShapeDtypeStruct((2 * NP, H), jnp.float32),
    )(*args)


_BN_S = 1.0 / jnp.sqrt(jnp.float32(1.0 + 1e-5))


def kernel(x_user, x_item, edge_index_user_to_item, edge_index_item_to_user,
           params):
    src_iu = edge_index_item_to_user[0] + NP
    dst_iu = edge_index_item_to_user[1]
    src_ui = edge_index_user_to_item[0]
    dst_ui = edge_index_user_to_item[1]

    def comb(a_iu, a_ui):
        return jnp.stack([a_iu.reshape(NS, EPW), a_ui.reshape(NS, EPW)],
                         axis=1).reshape(NW * EPW)

    src_c = comb(src_iu, src_ui)
    dst_c = comb(dst_iu, dst_ui)

    pad = ((0, NP - N), (0, 0))
    x2 = jnp.concatenate([jnp.pad(x_user, pad), jnp.pad(x_item, pad)])

    cnts = _seg_count(dst_c)
    cnt2 = cnts[:, :, :CWT].reshape(2 * NP, CWT)

    def folded(W, b, w2, b2):
        s = w2 * _BN_S
        return W.T * s[None, :], (b * s + b2)[None, :]

    au, cu = folded(*params["lin_in"]["user"], *params["bn_in"]["user"])
    ai, ci = folded(*params["lin_in"]["item"], *params["bn_in"]["item"])
    h2 = _in_proj(x2, jnp.stack([au, ai]), jnp.stack([cu, ci]))

    Wf, bf = params["final"]
    n_layers = len(params["layers"])
    for li, layer in enumerate(params["layers"]):
        agg = _seg_sum(h2, src_c, dst_c)
        als, ars, cvs = [], [], []
        for nt, conv_key in (("user", "item_to_user"),
                             ("item", "user_to_item")):
            Wl, bl, Wr = layer["conv"][conv_key]
            w2, b2 = layer["bn"][nt]
            s = w2 * _BN_S
            als.append(Wl.T * s[None, :])
            ars.append(Wr.T * s[None, :])
            cvs.append((bl * s + b2)[None, :])
        fin = (Wf.T, bf[None, :]) if li == n_layers - 1 else None
        h2 = _layer(agg.reshape(2 * NP, H), cnt2, h2,
                    jnp.stack(als), jnp.stack(ars), jnp.stack(cvs), fin)

    return (h2[:N], h2[NP:NP + N])
